# Initial kernel scaffold; baseline (speedup 1.0000x reference)
import jax
import jax.numpy as jnp
from jax.experimental import pallas as pl


def kernel(x, edge_index, W_node, W_edge, W_c1, b_c1, W_c2, b_c2, W_fuse, b_fuse, W_fc, b_fc, W_m1, b_m1, W_m2, b_m2, motif_bias, W_res, rms_scale, W_out):
    raise NotImplementedError("write your pallas kernel here")


# speedup vs baseline: 16.7091x; 16.7091x over previous
import jax
import jax.numpy as jnp
from jax import lax
from jax.experimental import pallas as pl
from jax.experimental.pallas import tpu as pltpu
from jax.experimental.pallas import tpu_sc as plsc

NC = 2
NS = 16
NW = NC * NS
L = 16

N = 10000
N_PAD = 10240
E = 320000
C = 80
CH = E // (NW * C)
RPT = N_PAD // NS


def _deg_body(dst3, onehot80, zeros640, out, iv, vbuf, shared, sem):
    c = lax.axis_index("c")
    s = lax.axis_index("s")
    wid = c * NS + s

    pltpu.sync_copy(zeros640, shared.at[pl.ds(s * RPT, RPT)])
    pltpu.sync_copy(onehot80, vbuf)
    pltpu.sync_copy(dst3.at[wid], iv)
    plsc.subcore_barrier()

    def scat(j, _):
        pltpu.sync_copy(vbuf, shared.at[iv.at[j]], add=True)
        return 0
    lax.fori_loop(0, CH, scat, 0)

    plsc.subcore_barrier()
    pltpu.sync_copy(shared.at[pl.ds(s * RPT, RPT)],
                    out.at[c, pl.ds(s * RPT, RPT)])


def _sc_deg(dst3, onehot80, zeros640):
    f = pl.kernel(
        _deg_body,
        out_type=jax.ShapeDtypeStruct((NC, N_PAD, 128), jnp.float32),
        mesh=plsc.VectorSubcoreMesh(core_axis_name="c", subcore_axis_name="s"),
        scratch_types=[
            pltpu.VMEM((CH, C), jnp.int32),
            pltpu.VMEM((C, 128), jnp.float32),
            pltpu.VMEM_SHARED((N_PAD, 128), jnp.float32),
            pltpu.SemaphoreType.DMA,
        ],
    )
    return f(dst3, onehot80, zeros640)


def _agg_body(src3, dst3, hs, zerosg, outg, ivs, ivd, rbuf, sharedg, sem):
    c = lax.axis_index("c")
    s = lax.axis_index("s")
    wid = c * NS + s

    pltpu.sync_copy(zerosg, sharedg.at[pl.ds(s * RPT, RPT)])
    pltpu.sync_copy(src3.at[wid], ivs)
    pltpu.sync_copy(dst3.at[wid], ivd)
    plsc.subcore_barrier()

    def step(j, _):
        pltpu.async_copy(hs.at[ivs.at[j]], rbuf, sem).wait()
        pltpu.sync_copy(rbuf, sharedg.at[ivd.at[j]], add=True)
        return 0
    lax.fori_loop(0, CH, step, 0)

    plsc.subcore_barrier()
    pltpu.sync_copy(sharedg.at[pl.ds(s * RPT, RPT)],
                    outg.at[c, pl.ds(s * RPT, RPT)])


def _sc_agg(src3, dst3, hs, zerosg):
    f = pl.kernel(
        _agg_body,
        out_type=jax.ShapeDtypeStruct((NC, N_PAD, 128), jnp.float32),
        mesh=plsc.VectorSubcoreMesh(core_axis_name="c", subcore_axis_name="s"),
        scratch_types=[
            pltpu.VMEM((CH, C), jnp.int32),
            pltpu.VMEM((CH, C), jnp.int32),
            pltpu.VMEM((C, 128), jnp.float32),
            pltpu.VMEM_SHARED((N_PAD, 128), jnp.float32),
            pltpu.SemaphoreType.DMA,
        ],
    )
    return f(src3, dst3, hs, zerosg)


def _t_body(src3, dst3, dinvrow, zeros640, outt, ivs, ivd, tbuf,
            sharedt, sem):
    c = lax.axis_index("c")
    s = lax.axis_index("s")
    wid = c * NS + s

    pltpu.sync_copy(zeros640, sharedt.at[pl.ds(s * RPT, RPT)])
    pltpu.sync_copy(src3.at[wid], ivs)
    pltpu.sync_copy(dst3.at[wid], ivd)
    plsc.subcore_barrier()

    def step(j, _):
        pltpu.async_copy(dinvrow.at[ivs.at[j]], tbuf, sem).wait()
        pltpu.sync_copy(tbuf, sharedt.at[ivd.at[j]], add=True)
        return 0
    lax.fori_loop(0, CH, step, 0)

    plsc.subcore_barrier()
    pltpu.sync_copy(sharedt.at[pl.ds(s * RPT, RPT)],
                    outt.at[c, pl.ds(s * RPT, RPT)])


def _sc_t(src3, dst3, dinvrow, zeros640):
    f = pl.kernel(
        _t_body,
        out_type=jax.ShapeDtypeStruct((NC, N_PAD, 128), jnp.float32),
        mesh=plsc.VectorSubcoreMesh(core_axis_name="c", subcore_axis_name="s"),
        scratch_types=[
            pltpu.VMEM((CH, C), jnp.int32),
            pltpu.VMEM((CH, C), jnp.int32),
            pltpu.VMEM((C, 128), jnp.float32),
            pltpu.VMEM_SHARED((N_PAD, 128), jnp.float32),
            pltpu.SemaphoreType.DMA,
        ],
    )
    return f(src3, dst3, dinvrow, zeros640)


def _h_body(x_ref, w_ref, o_ref):
    y = jnp.dot(x_ref[...], w_ref[...], preferred_element_type=jnp.float32)
    nrm = jnp.sqrt(jnp.sum(y * y, axis=-1, keepdims=True))
    o_ref[...] = y / jnp.maximum(nrm, 1e-12)


def _hs_body(deg_ref, h_ref, o_ref):
    degc = deg_ref[0] + deg_ref[1]
    deg0 = degc[:, 0:1]
    dinv = jnp.where(deg0 > 0, lax.rsqrt(jnp.maximum(deg0, 1e-12)), 0.0)
    o_ref[...] = h_ref[...] * dinv


def _dinvrow_body(deg_ref, o_ref):
    degc = deg_ref[0] + deg_ref[1]
    deg0 = degc[:, 0:1]
    dinv = jnp.where(deg0 > 0, lax.rsqrt(jnp.maximum(deg0, 1e-12)), 0.0)
    lane = lax.broadcasted_iota(jnp.int32, degc.shape, 1)
    o_ref[...] = jnp.where(lane == 0, dinv, 0.0)


def _tail_body(g_ref, t_ref, x_ref, we, wc1, bc1, wc2, bc2, wf, bf, wfc, bfc,
               wm1, bm1, wm2, bm2, mb, wres, rsc, wout, o_ref):
    G = g_ref[0] + g_ref[1]
    t = t_ref[0][:, 0:1] + t_ref[1][:, 0:1]
    tsafe = jnp.where(t > 0, t, 1.0)
    agg = jnp.where(t > 0, G / tsafe, 0.0)

    def dot(a, b):
        return jnp.dot(a, b, preferred_element_type=jnp.float32)

    m = dot(agg, we[...])
    z = jnp.maximum(dot(m, wc1[...]) + bc1[...], 0.0)
    z = dot(z, wc2[...]) + bc2[...]
    f = dot(z, wf[...]) + bf[...]
    g2 = dot(jnp.maximum(f, 0.0), wfc[...]) + bfc[...]
    u = jnp.maximum(dot(g2, wm1[...]) + bm1[...], 0.0)
    u = dot(u, wm2[...]) + bm2[...] + mb[...]
    u = u + dot(x_ref[...], wres[...])
    rms = jnp.sqrt(jnp.mean(u * u, axis=-1, keepdims=True) + 1e-6)
    u = (u / rms) * rsc[...]
    o_ref[...] = dot(u, wout[...])


BLK = 1000
GRID = N // BLK


def kernel(x, edge_index, W_node, W_edge, W_c1, b_c1, W_c2, b_c2,
           W_fuse, b_fuse, W_fc, b_fc, W_m1, b_m1, W_m2, b_m2,
           motif_bias, W_res, rms_scale, W_out):
    src3 = edge_index[0].reshape(NW, CH, C)
    dst3 = edge_index[1].reshape(NW, CH, C)

    onehot80 = jnp.tile(
        jnp.where(jnp.arange(128) == 0, 1.0, 0.0).astype(jnp.float32), (C, 1))
    zeros640 = jnp.zeros((RPT, 128), jnp.float32)
    deg2 = _sc_deg(dst3, onehot80, zeros640)

    h = pl.pallas_call(
        _h_body,
        grid=(GRID,),
        in_specs=[pl.BlockSpec((BLK, 128), lambda i: (i, 0)),
                  pl.BlockSpec((128, 128), lambda i: (0, 0))],
        out_specs=pl.BlockSpec((BLK, 128), lambda i: (i, 0)),
        out_shape=jax.ShapeDtypeStruct((N, 128), jnp.float32),
    )(x, W_node)

    hs = pl.pallas_call(
        _hs_body,
        grid=(GRID,),
        in_specs=[pl.BlockSpec((NC, BLK, 128), lambda i: (0, i, 0)),
                  pl.BlockSpec((BLK, 128), lambda i: (i, 0))],
        out_specs=pl.BlockSpec((BLK, 128), lambda i: (i, 0)),
        out_shape=jax.ShapeDtypeStruct((N, 128), jnp.float32),
    )(deg2, h)

    dinvrow = pl.pallas_call(
        _dinvrow_body,
        grid=(N_PAD // 1024,),
        in_specs=[pl.BlockSpec((NC, 1024, 128), lambda i: (0, i, 0))],
        out_specs=pl.BlockSpec((1024, 128), lambda i: (i, 0)),
        out_shape=jax.ShapeDtypeStruct((N_PAD, 128), jnp.float32),
    )(deg2)

    G2 = _sc_agg(src3, dst3, hs, zeros640)
    T2 = _sc_t(src3, dst3, dinvrow, zeros640)

    w2 = lambda a: a.reshape(1, -1)
    full = lambda shape: pl.BlockSpec(shape, lambda i: tuple(0 for _ in shape))

    logits = pl.pallas_call(
        _tail_body,
        grid=(GRID,),
        in_specs=[pl.BlockSpec((NC, BLK, 128), lambda i: (0, i, 0)),
                  pl.BlockSpec((NC, BLK, 128), lambda i: (0, i, 0)),
                  pl.BlockSpec((BLK, 128), lambda i: (i, 0)),
                  full((128, 128)), full((128, 128)), full((1, 128)),
                  full((128, 128)), full((1, 128)),
                  full((128, 128)), full((1, 128)),
                  full((128, 128)), full((1, 128)),
                  full((128, 128)), full((1, 128)),
                  full((128, 128)), full((1, 128)), full((1, 128)),
                  full((128, 128)), full((1, 128)),
                  full((128, 16))],
        out_specs=pl.BlockSpec((BLK, 16), lambda i: (i, 0)),
        out_shape=jax.ShapeDtypeStruct((N, 16), jnp.float32),
    )(G2, T2, x, W_edge, W_c1, w2(b_c1), W_c2, w2(b_c2), W_fuse, w2(b_fuse),
      W_fc, w2(b_fc), W_m1, w2(b_m1), W_m2, w2(b_m2), w2(motif_bias),
      W_res, w2(rms_scale), W_out)

    return logits


# speedup vs baseline: 32.4254x; 1.9406x over previous
import jax
import jax.numpy as jnp
from jax import lax
from jax.experimental import pallas as pl
from jax.experimental.pallas import tpu as pltpu
from jax.experimental.pallas import tpu_sc as plsc

NC = 2
NS = 16
NW = NC * NS
L = 16

N = 10000
N_PAD = 10240
E = 320000
C = 50
CH = E // (NW * C)
RPT = N_PAD // NS

_SC_PARAMS = pltpu.CompilerParams(use_tc_tiling_on_sc=False)


def _deg_body(dst3, onehot, zerosd, out, iv, vbuf, shared, sem):
    c = lax.axis_index("c")
    s = lax.axis_index("s")
    wid = c * NS + s

    pltpu.sync_copy(zerosd, shared.at[pl.ds(s * RPT, RPT)])
    pltpu.sync_copy(onehot, vbuf)
    pltpu.sync_copy(dst3.at[wid], iv)
    plsc.subcore_barrier()

    def scat(j, _):
        pltpu.sync_copy(vbuf, shared.at[iv.at[j]], add=True)
        return 0
    lax.fori_loop(0, CH, scat, 0)

    plsc.subcore_barrier()
    pltpu.sync_copy(shared.at[pl.ds(s * RPT, RPT)],
                    out.at[c, pl.ds(s * RPT, RPT)])


def _sc_deg(dst3, onehot, zerosd):
    f = pl.kernel(
        _deg_body,
        out_type=jax.ShapeDtypeStruct((NC, N_PAD, L), jnp.float32),
        mesh=plsc.VectorSubcoreMesh(core_axis_name="c", subcore_axis_name="s"),
        scratch_types=[
            pltpu.VMEM((CH, C), jnp.int32),
            pltpu.VMEM((C, L), jnp.float32),
            pltpu.VMEM_SHARED((N_PAD, L), jnp.float32),
            pltpu.SemaphoreType.DMA,
        ],
        compiler_params=_SC_PARAMS,
    )
    return f(dst3, onehot, zerosd)


def _agg_body(src3, dst3, hs, dinvrow, zerosg, zerost, outg, outt,
              ivs, ivd, rb0, rb1, tb0, tb1, sharedg, sharedt,
              sg0, sg1, st0, st1):
    c = lax.axis_index("c")
    s = lax.axis_index("s")
    wid = c * NS + s

    pltpu.sync_copy(zerosg, sharedg.at[pl.ds(s * RPT, RPT)])
    pltpu.sync_copy(zerost, sharedt.at[pl.ds(s * RPT, RPT)])
    pltpu.sync_copy(src3.at[wid], ivs)
    pltpu.sync_copy(dst3.at[wid], ivd)
    plsc.subcore_barrier()

    pltpu.async_copy(hs.at[ivs.at[0]], rb0, sg0)
    pltpu.async_copy(dinvrow.at[ivs.at[0]], tb0, st0)

    def step2(i, _):
        j0 = 2 * i
        pltpu.async_copy(hs.at[ivs.at[j0 + 1]], rb1, sg1)
        pltpu.async_copy(dinvrow.at[ivs.at[j0 + 1]], tb1, st1)
        pltpu.make_async_copy(hs.at[ivs.at[j0]], rb0, sg0).wait()
        pltpu.make_async_copy(dinvrow.at[ivs.at[j0]], tb0, st0).wait()
        pltpu.sync_copy(rb0, sharedg.at[ivd.at[j0]], add=True)
        pltpu.sync_copy(tb0, sharedt.at[ivd.at[j0]], add=True)

        @pl.when(j0 + 2 < CH)
        def _():
            pltpu.async_copy(hs.at[ivs.at[j0 + 2]], rb0, sg0)
            pltpu.async_copy(dinvrow.at[ivs.at[j0 + 2]], tb0, st0)

        pltpu.make_async_copy(hs.at[ivs.at[j0 + 1]], rb1, sg1).wait()
        pltpu.make_async_copy(dinvrow.at[ivs.at[j0 + 1]], tb1, st1).wait()
        pltpu.sync_copy(rb1, sharedg.at[ivd.at[j0 + 1]], add=True)
        pltpu.sync_copy(tb1, sharedt.at[ivd.at[j0 + 1]], add=True)
        return 0
    lax.fori_loop(0, CH // 2, step2, 0)

    plsc.subcore_barrier()
    pltpu.sync_copy(sharedg.at[pl.ds(s * RPT, RPT)],
                    outg.at[c, pl.ds(s * RPT, RPT)])
    pltpu.sync_copy(sharedt.at[pl.ds(s * RPT, RPT)],
                    outt.at[c, pl.ds(s * RPT, RPT)])


def _sc_agg(src3, dst3, hs, dinvrow, zerosg, zerost):
    f = pl.kernel(
        _agg_body,
        out_type=(jax.ShapeDtypeStruct((NC, N_PAD, 128), jnp.float32),
                  jax.ShapeDtypeStruct((NC, N_PAD, L), jnp.float32)),
        mesh=plsc.VectorSubcoreMesh(core_axis_name="c", subcore_axis_name="s"),
        scratch_types=[
            pltpu.VMEM((CH, C), jnp.int32),
            pltpu.VMEM((CH, C), jnp.int32),
            pltpu.VMEM((C, 128), jnp.float32),
            pltpu.VMEM((C, 128), jnp.float32),
            pltpu.VMEM((C, L), jnp.float32),
            pltpu.VMEM((C, L), jnp.float32),
            pltpu.VMEM_SHARED((N_PAD, 128), jnp.float32),
            pltpu.VMEM_SHARED((N_PAD, L), jnp.float32),
            pltpu.SemaphoreType.DMA,
            pltpu.SemaphoreType.DMA,
            pltpu.SemaphoreType.DMA,
            pltpu.SemaphoreType.DMA,
        ],
        compiler_params=_SC_PARAMS,
    )
    return f(src3, dst3, hs, dinvrow, zerosg, zerost)


def _h_body(x_ref, w_ref, o_ref):
    y = jnp.dot(x_ref[...], w_ref[...], preferred_element_type=jnp.float32)
    nrm = jnp.sqrt(jnp.sum(y * y, axis=-1, keepdims=True))
    o_ref[...] = y / jnp.maximum(nrm, 1e-12)


def _hs_body(deg_ref, h_ref, o_ref):
    degc = deg_ref[0] + deg_ref[1]
    deg0 = degc[:, 0:1]
    dinv = jnp.where(deg0 > 0, lax.rsqrt(jnp.maximum(deg0, 1e-12)), 0.0)
    o_ref[...] = h_ref[...] * dinv


def _dinvrow_body(deg_ref, o_ref):
    degc = deg_ref[0] + deg_ref[1]
    deg0 = degc[:, 0:1]
    dinv = jnp.where(deg0 > 0, lax.rsqrt(jnp.maximum(deg0, 1e-12)), 0.0)
    o_ref[...] = jnp.broadcast_to(dinv, degc.shape)


def _tail_body(g_ref, t_ref, x_ref, we, wc1, bc1, wc2, bc2, wf, bf, wfc, bfc,
               wm1, bm1, wm2, bm2, mb, wres, rsc, wout, o_ref):
    G = g_ref[0] + g_ref[1]
    t = t_ref[0][:, 0:1] + t_ref[1][:, 0:1]
    tsafe = jnp.where(t > 0, t, 1.0)
    agg = jnp.where(t > 0, G / tsafe, 0.0)

    def dot(a, b):
        return jnp.dot(a, b, preferred_element_type=jnp.float32)

    m = dot(agg, we[...])
    z = jnp.maximum(dot(m, wc1[...]) + bc1[...], 0.0)
    z = dot(z, wc2[...]) + bc2[...]
    f = dot(z, wf[...]) + bf[...]
    g2 = dot(jnp.maximum(f, 0.0), wfc[...]) + bfc[...]
    u = jnp.maximum(dot(g2, wm1[...]) + bm1[...], 0.0)
    u = dot(u, wm2[...]) + bm2[...] + mb[...]
    u = u + dot(x_ref[...], wres[...])
    rms = jnp.sqrt(jnp.mean(u * u, axis=-1, keepdims=True) + 1e-6)
    u = (u / rms) * rsc[...]
    o_ref[...] = dot(u, wout[...])


BLK = 1000
GRID = N // BLK


def kernel(x, edge_index, W_node, W_edge, W_c1, b_c1, W_c2, b_c2,
           W_fuse, b_fuse, W_fc, b_fc, W_m1, b_m1, W_m2, b_m2,
           motif_bias, W_res, rms_scale, W_out):
    src3 = edge_index[0].reshape(NW, CH, C)
    dst3 = edge_index[1].reshape(NW, CH, C)

    onehot = jnp.tile(
        jnp.where(jnp.arange(L) == 0, 1.0, 0.0).astype(jnp.float32), (C, 1))
    zerosd = jnp.zeros((RPT, L), jnp.float32)
    zerosg = jnp.zeros((RPT, 128), jnp.float32)
    deg2 = _sc_deg(dst3, onehot, zerosd)

    h = pl.pallas_call(
        _h_body,
        grid=(GRID,),
        in_specs=[pl.BlockSpec((BLK, 128), lambda i: (i, 0)),
                  pl.BlockSpec((128, 128), lambda i: (0, 0))],
        out_specs=pl.BlockSpec((BLK, 128), lambda i: (i, 0)),
        out_shape=jax.ShapeDtypeStruct((N, 128), jnp.float32),
    )(x, W_node)

    hs = pl.pallas_call(
        _hs_body,
        grid=(GRID,),
        in_specs=[pl.BlockSpec((NC, BLK, L), lambda i: (0, i, 0)),
                  pl.BlockSpec((BLK, 128), lambda i: (i, 0))],
        out_specs=pl.BlockSpec((BLK, 128), lambda i: (i, 0)),
        out_shape=jax.ShapeDtypeStruct((N, 128), jnp.float32),
    )(deg2, h)

    dinvrow = pl.pallas_call(
        _dinvrow_body,
        grid=(N_PAD // 1024,),
        in_specs=[pl.BlockSpec((NC, 1024, L), lambda i: (0, i, 0))],
        out_specs=pl.BlockSpec((1024, L), lambda i: (i, 0)),
        out_shape=jax.ShapeDtypeStruct((N_PAD, L), jnp.float32),
    )(deg2)

    G2, T2 = _sc_agg(src3, dst3, hs, dinvrow, zerosg, zerosd)

    w2 = lambda a: a.reshape(1, -1)
    full = lambda shape: pl.BlockSpec(shape, lambda i: tuple(0 for _ in shape))

    logits = pl.pallas_call(
        _tail_body,
        grid=(GRID,),
        in_specs=[pl.BlockSpec((NC, BLK, 128), lambda i: (0, i, 0)),
                  pl.BlockSpec((NC, BLK, L), lambda i: (0, i, 0)),
                  pl.BlockSpec((BLK, 128), lambda i: (i, 0)),
                  full((128, 128)), full((128, 128)), full((1, 128)),
                  full((128, 128)), full((1, 128)),
                  full((128, 128)), full((1, 128)),
                  full((128, 128)), full((1, 128)),
                  full((128, 128)), full((1, 128)),
                  full((128, 128)), full((1, 128)), full((1, 128)),
                  full((128, 128)), full((1, 128)),
                  full((128, 16))],
        out_specs=pl.BlockSpec((BLK, 16), lambda i: (i, 0)),
        out_shape=jax.ShapeDtypeStruct((N, 16), jnp.float32),
    )(G2, T2, x, W_edge, W_c1, w2(b_c1), W_c2, w2(b_c2), W_fuse, w2(b_fuse),
      W_fc, w2(b_fc), W_m1, w2(b_m1), W_m2, w2(b_m2), w2(motif_bias),
      W_res, w2(rms_scale), W_out)

    return logits


# speedup vs baseline: 32.4697x; 1.0014x over previous
import jax
import jax.numpy as jnp
from jax import lax
from jax.experimental import pallas as pl
from jax.experimental.pallas import tpu as pltpu
from jax.experimental.pallas import tpu_sc as plsc

NC = 2
NS = 16
NW = NC * NS
L = 16

N = 10000
N_PAD = 10240
E = 320000
C = 50
CH = E // (NW * C)
RPT = N_PAD // NS
N_ACC = N_PAD
RPA = N_ACC // NS

_SC_PARAMS = pltpu.CompilerParams(use_tc_tiling_on_sc=False)


def _deg_body(dst3, onehot, zerosd, out, iv, vbuf, shared, sem):
    c = lax.axis_index("c")
    s = lax.axis_index("s")
    wid = c * NS + s

    pltpu.sync_copy(zerosd, shared.at[pl.ds(s * RPT, RPT)])
    pltpu.sync_copy(onehot, vbuf)
    pltpu.sync_copy(dst3.at[wid], iv)
    plsc.subcore_barrier()

    def scat(j, _):
        pltpu.sync_copy(vbuf, shared.at[iv.at[j]], add=True)
        return 0
    lax.fori_loop(0, CH, scat, 0)

    plsc.subcore_barrier()
    pltpu.sync_copy(shared.at[pl.ds(s * RPT, RPT)],
                    out.at[c, pl.ds(s * RPT, RPT)])


def _sc_deg(dst3, onehot, zerosd):
    f = pl.kernel(
        _deg_body,
        out_type=jax.ShapeDtypeStruct((NC, N_PAD, L), jnp.float32),
        mesh=plsc.VectorSubcoreMesh(core_axis_name="c", subcore_axis_name="s"),
        scratch_types=[
            pltpu.VMEM((CH, C), jnp.int32),
            pltpu.VMEM((C, L), jnp.float32),
            pltpu.VMEM_SHARED((N_PAD, L), jnp.float32),
            pltpu.SemaphoreType.DMA,
        ],
        compiler_params=_SC_PARAMS,
    )
    return f(dst3, onehot, zerosd)


def _agg_body(src3, dst3, hs, dinvrow, zerosg, zerost, outg, outt,
              ivs, ivd, rb0, rb1, tb0, tb1, sharedg, sharedt,
              sg0, sg1, st0, st1):
    c = lax.axis_index("c")
    s = lax.axis_index("s")
    wid = c * NS + s

    pltpu.sync_copy(zerosg, sharedg.at[pl.ds(s * RPA, RPA)])
    pltpu.sync_copy(zerost, sharedt.at[pl.ds(s * RPA, RPA)])
    pltpu.sync_copy(src3.at[wid], ivs)
    pltpu.sync_copy(dst3.at[wid], ivd)
    plsc.subcore_barrier()

    pltpu.async_copy(hs.at[ivs.at[0]], rb0, sg0)
    pltpu.async_copy(dinvrow.at[ivs.at[0]], tb0, st0)

    def step2(i, _):
        j0 = 2 * i
        pltpu.async_copy(hs.at[ivs.at[j0 + 1]], rb1, sg1)
        pltpu.async_copy(dinvrow.at[ivs.at[j0 + 1]], tb1, st1)
        pltpu.make_async_copy(hs.at[ivs.at[j0]], rb0, sg0).wait()
        pltpu.make_async_copy(dinvrow.at[ivs.at[j0]], tb0, st0).wait()
        pltpu.sync_copy(rb0, sharedg.at[ivd.at[j0]], add=True)
        pltpu.sync_copy(tb0, sharedt.at[ivd.at[j0]], add=True)

        @pl.when(j0 + 2 < CH)
        def _():
            pltpu.async_copy(hs.at[ivs.at[j0 + 2]], rb0, sg0)
            pltpu.async_copy(dinvrow.at[ivs.at[j0 + 2]], tb0, st0)

        pltpu.make_async_copy(hs.at[ivs.at[j0 + 1]], rb1, sg1).wait()
        pltpu.make_async_copy(dinvrow.at[ivs.at[j0 + 1]], tb1, st1).wait()
        pltpu.sync_copy(rb1, sharedg.at[ivd.at[j0 + 1]], add=True)
        pltpu.sync_copy(tb1, sharedt.at[ivd.at[j0 + 1]], add=True)
        return 0
    lax.fori_loop(0, CH // 2, step2, 0)

    plsc.subcore_barrier()
    pltpu.sync_copy(sharedg.at[pl.ds(s * RPA, RPA)],
                    outg.at[c, pl.ds(s * RPA, RPA)])
    pltpu.sync_copy(sharedt.at[pl.ds(s * RPA, RPA)],
                    outt.at[c, pl.ds(s * RPA, RPA)])


def _sc_agg(src3, dst3, hs, dinvrow, zerosg, zerost):
    f = pl.kernel(
        _agg_body,
        out_type=(jax.ShapeDtypeStruct((NC, N_ACC, 128), jnp.float32),
                  jax.ShapeDtypeStruct((NC, N_ACC, L), jnp.float32)),
        mesh=plsc.VectorSubcoreMesh(core_axis_name="c", subcore_axis_name="s"),
        scratch_types=[
            pltpu.VMEM((CH, C), jnp.int32),
            pltpu.VMEM((CH, C), jnp.int32),
            pltpu.VMEM((C, 128), jnp.float32),
            pltpu.VMEM((C, 128), jnp.float32),
            pltpu.VMEM((C, L), jnp.float32),
            pltpu.VMEM((C, L), jnp.float32),
            pltpu.VMEM_SHARED((N_ACC, 128), jnp.float32),
            pltpu.VMEM_SHARED((N_ACC, L), jnp.float32),
            pltpu.SemaphoreType.DMA,
            pltpu.SemaphoreType.DMA,
            pltpu.SemaphoreType.DMA,
            pltpu.SemaphoreType.DMA,
        ],
        compiler_params=_SC_PARAMS,
    )
    return f(src3, dst3, hs, dinvrow, zerosg, zerost)


def _h_body(x_ref, w_ref, o_ref):
    y = jnp.dot(x_ref[...], w_ref[...], preferred_element_type=jnp.float32)
    nrm = jnp.sqrt(jnp.sum(y * y, axis=-1, keepdims=True))
    o_ref[...] = y / jnp.maximum(nrm, 1e-12)


def _hs_body(deg_ref, h_ref, o_ref):
    degc = deg_ref[0] + deg_ref[1]
    deg0 = degc[:, 0:1]
    dinv = jnp.where(deg0 > 0, lax.rsqrt(jnp.maximum(deg0, 1e-12)), 0.0)
    o_ref[...] = h_ref[...] * dinv


def _dinvrow_body(deg_ref, o_ref):
    degc = deg_ref[0] + deg_ref[1]
    deg0 = degc[:, 0:1]
    dinv = jnp.where(deg0 > 0, lax.rsqrt(jnp.maximum(deg0, 1e-12)), 0.0)
    o_ref[...] = jnp.broadcast_to(dinv, degc.shape)


def _tail_body(g_ref, t_ref, x_ref, we, wc1, bc1, wc2, bc2, wf, bf, wfc, bfc,
               wm1, bm1, wm2, bm2, mb, wres, rsc, wout, o_ref):
    G = g_ref[0] + g_ref[1]
    t = t_ref[0][:, 0:1] + t_ref[1][:, 0:1]
    tsafe = jnp.where(t > 0, t, 1.0)
    agg = jnp.where(t > 0, G / tsafe, 0.0)

    def dot(a, b):
        return jnp.dot(a, b, preferred_element_type=jnp.float32)

    m = dot(agg, we[...])
    z = jnp.maximum(dot(m, wc1[...]) + bc1[...], 0.0)
    z = dot(z, wc2[...]) + bc2[...]
    f = dot(z, wf[...]) + bf[...]
    g2 = dot(jnp.maximum(f, 0.0), wfc[...]) + bfc[...]
    u = jnp.maximum(dot(g2, wm1[...]) + bm1[...], 0.0)
    u = dot(u, wm2[...]) + bm2[...] + mb[...]
    u = u + dot(x_ref[...], wres[...])
    rms = jnp.sqrt(jnp.mean(u * u, axis=-1, keepdims=True) + 1e-6)
    u = (u / rms) * rsc[...]
    o_ref[...] = dot(u, wout[...])


BLK = 1000
GRID = N // BLK


def kernel(x, edge_index, W_node, W_edge, W_c1, b_c1, W_c2, b_c2,
           W_fuse, b_fuse, W_fc, b_fc, W_m1, b_m1, W_m2, b_m2,
           motif_bias, W_res, rms_scale, W_out):
    src3 = edge_index[0].reshape(NW, CH, C)
    dst3 = edge_index[1].reshape(NW, CH, C)

    onehot = jnp.tile(
        jnp.where(jnp.arange(L) == 0, 1.0, 0.0).astype(jnp.float32), (C, 1))
    zerosd = jnp.zeros((RPT, L), jnp.float32)
    zerosg = jnp.zeros((RPA, 128), jnp.float32)
    zerost = jnp.zeros((RPA, L), jnp.float32)
    deg2 = _sc_deg(dst3, onehot, zerosd)

    h = pl.pallas_call(
        _h_body,
        grid=(GRID,),
        in_specs=[pl.BlockSpec((BLK, 128), lambda i: (i, 0)),
                  pl.BlockSpec((128, 128), lambda i: (0, 0))],
        out_specs=pl.BlockSpec((BLK, 128), lambda i: (i, 0)),
        out_shape=jax.ShapeDtypeStruct((N, 128), jnp.float32),
    )(x, W_node)

    hs = pl.pallas_call(
        _hs_body,
        grid=(GRID,),
        in_specs=[pl.BlockSpec((NC, BLK, L), lambda i: (0, i, 0)),
                  pl.BlockSpec((BLK, 128), lambda i: (i, 0))],
        out_specs=pl.BlockSpec((BLK, 128), lambda i: (i, 0)),
        out_shape=jax.ShapeDtypeStruct((N, 128), jnp.float32),
    )(deg2, h)

    dinvrow = pl.pallas_call(
        _dinvrow_body,
        grid=(N_PAD // 1024,),
        in_specs=[pl.BlockSpec((NC, 1024, L), lambda i: (0, i, 0))],
        out_specs=pl.BlockSpec((1024, L), lambda i: (i, 0)),
        out_shape=jax.ShapeDtypeStruct((N_PAD, L), jnp.float32),
    )(deg2)

    G2, T2 = _sc_agg(src3, dst3, hs, dinvrow, zerosg, zerost)

    w2 = lambda a: a.reshape(1, -1)
    full = lambda shape: pl.BlockSpec(shape, lambda i: tuple(0 for _ in shape))

    logits = pl.pallas_call(
        _tail_body,
        grid=(GRID,),
        in_specs=[pl.BlockSpec((NC, BLK, 128), lambda i: (0, i, 0)),
                  pl.BlockSpec((NC, BLK, L), lambda i: (0, i, 0)),
                  pl.BlockSpec((BLK, 128), lambda i: (i, 0)),
                  full((128, 128)), full((128, 128)), full((1, 128)),
                  full((128, 128)), full((1, 128)),
                  full((128, 128)), full((1, 128)),
                  full((128, 128)), full((1, 128)),
                  full((128, 128)), full((1, 128)),
                  full((128, 128)), full((1, 128)), full((1, 128)),
                  full((128, 128)), full((1, 128)),
                  full((128, 16))],
        out_specs=pl.BlockSpec((BLK, 16), lambda i: (i, 0)),
        out_shape=jax.ShapeDtypeStruct((N, 16), jnp.float32),
    )(G2, T2, x, W_edge, W_c1, w2(b_c1), W_c2, w2(b_c2), W_fuse, w2(b_fuse),
      W_fc, w2(b_fc), W_m1, w2(b_m1), W_m2, w2(b_m2), w2(motif_bias),
      W_res, w2(rms_scale), W_out)

    return logits
